# Initial kernel scaffold; baseline (speedup 1.0000x reference)
import jax
import jax.numpy as jnp
from jax.experimental import pallas as pl


def kernel(x):
    raise NotImplementedError("write your pallas kernel here")


# speedup vs baseline: 2.3729x; 2.3729x over previous
import jax
import jax.numpy as jnp
from jax.experimental import pallas as pl
from jax.experimental.pallas import tpu as pltpu

_W1 = 4096
_W2 = 8192


def _argmax_kernel(x_ref, idx_ref, rmax_ref, ridx_ref):
    j = pl.program_id(0)
    xb = x_ref[...]
    bmax = jnp.max(xb, axis=1, keepdims=True)
    iota = jax.lax.broadcasted_iota(jnp.int32, xb.shape, 1)
    bidx = jnp.min(
        jnp.where(xb == bmax, iota, xb.shape[1]), axis=1, keepdims=True
    ) + j * _W1

    @pl.when(j == 0)
    def _():
        rmax_ref[...] = bmax
        ridx_ref[...] = bidx

    @pl.when(j > 0)
    def _():
        upd = bmax > rmax_ref[...]
        ridx_ref[...] = jnp.where(upd, bidx, ridx_ref[...])
        rmax_ref[...] = jnp.maximum(bmax, rmax_ref[...])

    @pl.when(j == pl.num_programs(0) - 1)
    def _():
        idx_ref[...] = ridx_ref[...]


def _onehot_kernel(idx_ref, out_ref):
    j = pl.program_id(0)
    iota = jax.lax.broadcasted_iota(jnp.int32, out_ref.shape, 1) + j * _W2
    out_ref[...] = (iota == idx_ref[...]).astype(jnp.float32)


def kernel(x):
    rows, cols = x.shape
    idx = pl.pallas_call(
        _argmax_kernel,
        grid=(cols // _W1,),
        in_specs=[pl.BlockSpec((rows, _W1), lambda j: (0, j))],
        out_specs=pl.BlockSpec((rows, 1), lambda j: (0, 0)),
        out_shape=jax.ShapeDtypeStruct((rows, 1), jnp.int32),
        scratch_shapes=[
            pltpu.VMEM((rows, 1), jnp.float32),
            pltpu.VMEM((rows, 1), jnp.int32),
        ],
    )(x)
    out = pl.pallas_call(
        _onehot_kernel,
        grid=(cols // _W2,),
        in_specs=[pl.BlockSpec((rows, 1), lambda j: (0, 0))],
        out_specs=pl.BlockSpec((rows, _W2), lambda j: (0, j)),
        out_shape=jax.ShapeDtypeStruct((rows, cols), jnp.float32),
    )(idx)
    return out
